# baseline (speedup 1.0000x reference)
import jax
import jax.numpy as jnp
from jax.experimental import pallas as pl
from jax.experimental.pallas import tpu as pltpu

_JITTER_EPS = 0.1
_TOK_BLK = 512
_NBUF = 4


def _mixer_body(lg_ref, om_ref, sample_ref, out_hbm, buf, sems):
    i = pl.program_id(0)
    slot = jax.lax.rem(i, _NBUF)
    n_blk = pl.num_programs(0)
    tok = lg_ref.shape[0]

    def _copy(s, blk):
        return pltpu.make_async_copy(
            buf.at[s], out_hbm.at[pl.ds(blk * tok, tok), :], sems.at[s]
        )

    @pl.when(i >= _NBUF)
    def _():
        _copy(slot, i - _NBUF).wait()

    lg = lg_ref[...]
    mx = jnp.max(lg, axis=-1, keepdims=True)
    ids = jax.lax.broadcasted_iota(jnp.int32, lg.shape, 1)
    amax = jnp.min(jnp.where(lg == mx, ids, lg.shape[1]), axis=-1, keepdims=True)
    factor = jnp.maximum(jnp.abs(lg), mx)
    mask = (mx - lg) / factor > 2.0 * _JITTER_EPS
    e = jnp.where(mask, 0.0, jnp.exp(lg - mx))
    m = 1.0 / jnp.sum(e, axis=-1, keepdims=True)
    sample_ref[...] = amax
    buf[slot] = m * om_ref[...][None, :]
    _copy(slot, i).start()

    @pl.when(i == n_blk - 1)
    def _():
        for s in range(_NBUF):
            _copy(s, i).wait()


def kernel(logits, omega):
    n_tok, n_exp = logits.shape
    dim = omega.shape[0]
    grid = (n_tok // _TOK_BLK,)
    sample, multiplier = pl.pallas_call(
        _mixer_body,
        grid=grid,
        in_specs=[
            pl.BlockSpec((_TOK_BLK, n_exp), lambda i: (i, 0)),
            pl.BlockSpec((dim,), lambda i: (0,)),
        ],
        out_specs=[
            pl.BlockSpec((_TOK_BLK, 1), lambda i: (i, 0)),
            pl.BlockSpec(memory_space=pl.ANY),
        ],
        out_shape=[
            jax.ShapeDtypeStruct((n_tok, 1), jnp.int32),
            jax.ShapeDtypeStruct((n_tok, dim), jnp.float32),
        ],
        scratch_shapes=[
            pltpu.VMEM((_NBUF, _TOK_BLK, dim), jnp.float32),
            pltpu.SemaphoreType.DMA((_NBUF,)),
        ],
    )(logits, omega)
    return sample, multiplier, jnp.float32(0.0)
